# baseline (speedup 1.0000x reference)
import functools

import jax
import jax.numpy as jnp
from jax import lax
from jax.experimental import pallas as pl
from jax.experimental.pallas import tpu as pltpu
from jax.experimental.pallas import tpu_sc as plsc

M = 16
DH = 128
DX = 128
N = 32768
BLK = 2048
NEG = -1e30

NWORK = 32
CHUNK = 256
CPW = N // (NWORK * CHUNK)
NCHUNK = N // CHUNK
LANES = 16


def _scores_body(h_ref, a_ref, x_ref, seg_ref, s_ref, ha_ref):
    i = pl.program_id(0)

    @pl.when(i == 0)
    def _init():
        ha_ref[...] = jnp.dot(h_ref[...], a_ref[...],
                              preferred_element_type=jnp.float32)

    x_blk = x_ref[...]
    seg = seg_ref[0]
    scores = lax.dot_general(ha_ref[...], x_blk,
                             (((1,), (1,)), ((), ())),
                             preferred_element_type=jnp.float32)
    seg_iota = lax.broadcasted_iota(jnp.int32, (M, BLK), 0)
    onehot = seg == seg_iota
    s = jnp.sum(jnp.where(onehot, scores, 0.0), axis=0)
    s_ref[...] = s.reshape(1, 1, BLK)


def _tc_scores(h, a, x, seg3):
    nb = N // BLK
    return pl.pallas_call(
        _scores_body,
        grid=(nb,),
        in_specs=[
            pl.BlockSpec((M, DH), lambda i: (0, 0)),
            pl.BlockSpec((DH, DX), lambda i: (0, 0)),
            pl.BlockSpec((BLK, DX), lambda i: (i, 0)),
            pl.BlockSpec((1, 1, BLK), lambda i: (i, 0, 0)),
        ],
        out_specs=pl.BlockSpec((1, 1, BLK), lambda i: (i, 0, 0)),
        out_shape=jax.ShapeDtypeStruct((nb, 1, BLK), jnp.float32),
        scratch_shapes=[pltpu.VMEM((M, DX), jnp.float32)],
    )(h, a, x, seg3)


def _lane_perm(v, idx):
    return lax.gather(
        v, idx[:, None],
        lax.GatherDimensionNumbers(offset_dims=(), collapsed_slice_dims=(0,),
                                   start_index_map=(0,)),
        slice_sizes=(1,),
        mode=lax.GatherScatterMode.PROMISE_IN_BOUNDS)


def _bfly_max(v, lane_iota):
    for sh in (8, 4, 2, 1):
        v = jnp.maximum(v, _lane_perm(v, lane_iota ^ sh))
    return v


def _bfly_sum(v, lane_iota):
    for sh in (8, 4, 2, 1):
        v = v + _lane_perm(v, lane_iota ^ sh)
    return v


def _sc_body(x_hbm, s_hbm, seg_hbm, maxs_hbm, zs_hbm, nums_hbm,
             x_v, s_v, seg_v, e_v, cm_v, zv_v, acc_v):
    wid = lax.axis_index("s") * 2 + lax.axis_index("c")
    nvec = CHUNK // LANES
    lane_iota = lax.iota(jnp.int32, LANES)

    def chunk_body(c, _):
        cid = wid * CPW + c
        base = cid * CHUNK
        pltpu.sync_copy(x_hbm.at[pl.ds(base, CHUNK)], x_v)
        pltpu.sync_copy(s_hbm.at[pl.ds(base, CHUNK)], s_v)
        pltpu.sync_copy(seg_hbm.at[pl.ds(base, CHUNK)], seg_v)

        lo = seg_v[pl.ds(0, LANES)][0]
        hi = seg_v[pl.ds(CHUNK - LANES, LANES)][LANES - 1]

        def max_loop(m, cm):
            def vloop(v, acc):
                sv = s_v[pl.ds(v * LANES, LANES)]
                gv = seg_v[pl.ds(v * LANES, LANES)]
                return jnp.maximum(acc, jnp.where(gv == m, sv, NEG))
            mv = lax.fori_loop(0, nvec, vloop,
                               jnp.full((LANES,), NEG, jnp.float32))
            return jnp.where(lane_iota == m, _bfly_max(mv, lane_iota), cm)
        cm = lax.fori_loop(lo, hi + 1, max_loop,
                           jnp.full((LANES,), NEG, jnp.float32))
        cm_v[...] = cm

        def e_loop(v, _):
            sv = s_v[pl.ds(v * LANES, LANES)]
            gv = seg_v[pl.ds(v * LANES, LANES)]
            mg = plsc.load_gather(cm_v, [gv])
            e_v[pl.ds(v * LANES, LANES)] = jnp.exp(sv - mg)
            return 0
        lax.fori_loop(0, nvec, e_loop, 0)

        def z_loop(m, zacc):
            def vloop(v, acc):
                ev = e_v[pl.ds(v * LANES, LANES)]
                gv = seg_v[pl.ds(v * LANES, LANES)]
                return acc + jnp.where(gv == m, ev, 0.0)
            zv = lax.fori_loop(0, nvec, vloop, jnp.zeros((LANES,), jnp.float32))
            return jnp.where(lane_iota == m, _bfly_sum(zv, lane_iota), zacc)
        zvec = lax.fori_loop(lo, hi + 1, z_loop, jnp.zeros((LANES,), jnp.float32))
        zv_v[...] = zvec

        def zero_loop(d, _):
            acc_v[pl.ds(d * LANES, LANES)] = jnp.zeros((LANES,), jnp.float32)
            return 0
        lax.fori_loop(0, (M * DX) // LANES, zero_loop, 0)

        def node_loop(v, _):
            ev = e_v[pl.ds(v * LANES, LANES)]
            gv = seg_v[pl.ds(v * LANES, LANES)]
            for l in range(LANES):
                ej = ev[l]
                rowbase = gv[l] * DX
                for k in range(DX // LANES):
                    sl = pl.ds(rowbase + k * LANES, LANES)
                    acc_v[sl] = acc_v[sl] + ej * x_v[v * LANES + l,
                                                     pl.ds(k * LANES, LANES)]
            return 0
        lax.fori_loop(0, nvec, node_loop, 0)

        pltpu.sync_copy(cm_v, maxs_hbm.at[cid])
        pltpu.sync_copy(zv_v, zs_hbm.at[cid])
        pltpu.sync_copy(acc_v, nums_hbm.at[cid])
        return 0

    lax.fori_loop(0, CPW, chunk_body, 0)


def _sc_partials(x, s, seg):
    mesh = plsc.VectorSubcoreMesh(core_axis_name="c", subcore_axis_name="s")
    f = pl.kernel(
        _sc_body,
        out_type=[
            jax.ShapeDtypeStruct((NCHUNK, LANES), jnp.float32),
            jax.ShapeDtypeStruct((NCHUNK, LANES), jnp.float32),
            jax.ShapeDtypeStruct((NCHUNK, M * DX), jnp.float32),
        ],
        mesh=mesh,
        compiler_params=pltpu.CompilerParams(needs_layout_passes=False),
        scratch_types=[
            pltpu.VMEM((CHUNK, DX), jnp.float32),
            pltpu.VMEM((CHUNK,), jnp.float32),
            pltpu.VMEM((CHUNK,), jnp.int32),
            pltpu.VMEM((CHUNK,), jnp.float32),
            pltpu.VMEM((LANES,), jnp.float32),
            pltpu.VMEM((LANES,), jnp.float32),
            pltpu.VMEM((M * DX,), jnp.float32),
        ],
    )
    return f(x, s, seg)


def _combine_body(maxs_ref, zs_ref, nums_ref, out_ref):
    maxs = maxs_ref[...]
    zs = zs_ref[...]
    gmax = jnp.max(maxs, axis=0, keepdims=True)
    scale = jnp.exp(maxs - gmax)
    zz = jnp.sum(zs * scale, axis=0)
    nums = nums_ref[...].reshape(NCHUNK, M, DX)
    wnum = jnp.sum(nums * scale[:, :, None], axis=0)
    z_col = zz.reshape(M, 1)
    out_ref[...] = jnp.where(z_col > 0.0, wnum / z_col, 0.0)


def _tc_combine(maxs, zs, nums):
    return pl.pallas_call(
        _combine_body,
        out_shape=jax.ShapeDtypeStruct((M, DX), jnp.float32),
    )(maxs, zs, nums)


def kernel(h, x, segment_ids, a):
    nb = N // BLK
    seg3 = segment_ids.reshape(nb, 1, BLK)
    s3 = _tc_scores(h, a, x, seg3)
    s = s3.reshape(N)
    maxs, zs, nums = _sc_partials(x, s, segment_ids)
    return _tc_combine(maxs, zs, nums)
